# Initial kernel scaffold; baseline (speedup 1.0000x reference)
import functools

import jax
import jax.numpy as jnp
from jax import lax
from jax.experimental import pallas as pl
from jax.experimental.pallas import tpu as pltpu
from jax.experimental.pallas import tpu_sc as plsc

D = 128
NC = 2
NS = 16
LANES = 16
CHUNK = 14256
ALLOC = 14336
TRASH = 14320
G = 128
W = 2048
FLUSH_AT = G - LANES


def _sc_segment_sum(x, src, dst, ew, n_nodes):
    e_pad = src.shape[0]
    ept = e_pad // NS
    n_win = ept // W
    n_chunks = -(-n_nodes // CHUNK)
    chunks_per_core = n_chunks // NC

    mesh = plsc.VectorSubcoreMesh(core_axis_name="c", subcore_axis_name="s")

    @functools.partial(
        pl.kernel,
        out_type=jax.ShapeDtypeStruct((n_nodes, D), jnp.float32),
        mesh=mesh,
        scratch_types=[
            pltpu.VMEM_SHARED((ALLOC, D), jnp.float32),
            pltpu.VMEM((W,), jnp.int32),
            pltpu.VMEM((W,), jnp.int32),
            pltpu.VMEM((W,), jnp.float32),
            pltpu.VMEM((1, G), jnp.int32),
            pltpu.VMEM((1, G), jnp.int32),
            pltpu.VMEM((1, G), jnp.float32),
            pltpu.VMEM((G, D), jnp.float32),
            pltpu.VMEM((64, D), jnp.float32),
        ],
    )
    def seg_kernel(x_hbm, src_hbm, dst_hbm, ew_hbm, out_hbm,
                   acc, wsrc, wdst, wew, csrc, cdst, cew, rows, zbuf):
        c = lax.axis_index("c")
        s = lax.axis_index("s")
        wid = s * NC + c

        zero16f = jnp.zeros((LANES,), jnp.float32)
        iota16 = jnp.arange(LANES, dtype=jnp.int32)
        zero16i = jnp.zeros((LANES,), jnp.int32)

        @pl.loop(0, 64)
        def _(r):
            for l in range(D // LANES):
                zbuf[r, pl.ds(l * LANES, LANES)] = zero16f

        def refill():
            @pl.loop(0, G, step=LANES)
            def _(i):
                csrc[0, pl.ds(i, LANES)] = wid * G + i + iota16
                cdst[0, pl.ds(i, LANES)] = TRASH + (iota16 & 7)
                cew[0, pl.ds(i, LANES)] = zero16f

        def flush():
            pltpu.sync_copy(x_hbm.at[csrc.at[0]], rows)

            @pl.loop(0, G)
            def _(r):
                sval = cew[0, r]
                for l in range(D // LANES):
                    rows[r, pl.ds(l * LANES, LANES)] = (
                        rows[r, pl.ds(l * LANES, LANES)] * sval)

            pltpu.sync_copy(rows, acc.at[cdst.at[0]], add=True)
            refill()

        for phase in range(chunks_per_core):
            chunk = c * chunks_per_core + phase
            lo = chunk * CHUNK
            hi = jnp.minimum(lo + CHUNK, n_nodes)

            @pl.loop(0, ALLOC // NS // 64)
            def _(k):
                acc_off = pl.multiple_of(s * (ALLOC // NS) + k * 64, 64)
                pltpu.sync_copy(zbuf, acc.at[pl.ds(acc_off, 64)])
            refill()
            plsc.subcore_barrier()

            def win_body(w, count):
                base = pl.multiple_of(s * ept + w * W, W)
                pltpu.sync_copy(src_hbm.at[pl.ds(base, W)], wsrc)
                pltpu.sync_copy(dst_hbm.at[pl.ds(base, W)], wdst)
                pltpu.sync_copy(ew_hbm.at[pl.ds(base, W)], wew)

                def slice_body(i, cnt):
                    off = pl.multiple_of(i * LANES, LANES)
                    sv = wsrc[pl.ds(off, LANES)]
                    dv = wdst[pl.ds(off, LANES)]
                    wv = wew[pl.ds(off, LANES)]
                    m = (dv >= lo) & (dv < hi)
                    mi = m.astype(jnp.int32)
                    cs = plsc.cumsum(mi)
                    pos = cnt + cs - mi
                    plsc.store_scatter(csrc, [zero16i, pos], sv, mask=m)
                    plsc.store_scatter(cdst, [zero16i, pos], dv - lo, mask=m)
                    plsc.store_scatter(cew, [zero16i, pos], wv, mask=m)
                    cnt = cnt + jnp.sum(mi)

                    @pl.when(cnt >= FLUSH_AT)
                    def _():
                        flush()

                    return jnp.where(cnt >= FLUSH_AT, 0, cnt)

                return lax.fori_loop(0, W // LANES, slice_body, count)

            count = lax.fori_loop(0, n_win, win_body, jnp.int32(0))
            flush()
            plsc.subcore_barrier()

            rc = hi - lo
            full_rows = 888
            row0 = pl.multiple_of(s * full_rows, 8)
            pltpu.sync_copy(acc.at[pl.ds(row0, full_rows)],
                            out_hbm.at[pl.ds(lo + row0, full_rows)])
            tail_base = full_rows * NS

            @pl.when(tail_base + s * 8 < rc)
            def _():
                t0 = pl.multiple_of(tail_base + s * 8, 8)
                pltpu.sync_copy(acc.at[pl.ds(t0, 8)],
                                out_hbm.at[pl.ds(lo + t0, 8)])

            plsc.subcore_barrier()

    return seg_kernel(x, src, dst, ew)


def _tc_linear(aggr, x, w_rel_t, w_root_t, b2d):
    n = aggr.shape[0]
    blk = 1000
    grid = (n // blk,)

    def body(a_ref, x_ref, wr_ref, wq_ref, b_ref, o_ref):
        o_ref[...] = (
            jnp.dot(a_ref[...], wr_ref[...], preferred_element_type=jnp.float32)
            + jnp.dot(x_ref[...], wq_ref[...], preferred_element_type=jnp.float32)
            + b_ref[...])

    return pl.pallas_call(
        body,
        grid=grid,
        in_specs=[
            pl.BlockSpec((blk, D), lambda i: (i, 0)),
            pl.BlockSpec((blk, D), lambda i: (i, 0)),
            pl.BlockSpec((D, D), lambda i: (0, 0)),
            pl.BlockSpec((D, D), lambda i: (0, 0)),
            pl.BlockSpec((1, D), lambda i: (0, 0)),
        ],
        out_specs=pl.BlockSpec((blk, D), lambda i: (i, 0)),
        out_shape=jax.ShapeDtypeStruct((n, D), jnp.float32),
    )(aggr, x, w_rel_t, w_root_t, b2d)


def kernel(x, edge_index, edge_weights, W_rel, b_rel, W_root):
    n_nodes = x.shape[0]
    n_elec = 19
    repeat = n_nodes // n_elec
    ew_full = jnp.tile(edge_weights, repeat)

    src = edge_index[0]
    dst = edge_index[1]
    e = src.shape[0]
    e_pad = -(-e // (NS * W)) * (NS * W)
    pad = e_pad - e
    src_p = jnp.concatenate([src, jnp.zeros((pad,), jnp.int32)])
    dst_p = jnp.concatenate([dst, jnp.full((pad,), -1, jnp.int32)])
    ew_p = jnp.concatenate([ew_full, jnp.zeros((pad,), jnp.float32)])

    aggr = _sc_segment_sum(x, src_p, dst_p, ew_p, n_nodes)
    return _tc_linear(aggr, x, W_rel.T, W_root.T, b_rel[None, :])


# speedup vs baseline: 6.5766x; 6.5766x over previous
import dataclasses
import functools

import jax
import jax.numpy as jnp
from jax import lax
from jax.experimental import pallas as pl
from jax.experimental.pallas import tpu as pltpu
from jax.experimental.pallas import tpu_sc as plsc

D = 128
NC = 2
NS = 16
LANES = 16
CHUNK = 9504
ALLOC = 10240
TRASH = 10232
G = 128
W = 2048
FLUSH_AT = G - LANES


def _sc_segment_sum(x, src, dst, ew, n_nodes):
    e_pad = src.shape[0]
    ept = e_pad // NS
    n_win = ept // W
    n_chunks = -(-n_nodes // CHUNK)
    chunks_per_core = n_chunks // NC
    min_chunk = n_nodes - (n_chunks - 1) * CHUNK
    full_rows = (min_chunk // NS) // 8 * 8
    assert CHUNK - full_rows * NS <= NS * 8

    mesh = plsc.VectorSubcoreMesh(core_axis_name="c", subcore_axis_name="s")
    cp = pltpu.CompilerParams()
    if "needs_layout_passes" in pltpu.CompilerParams.__dataclass_fields__:
        cp = dataclasses.replace(cp, needs_layout_passes=False)

    @functools.partial(
        pl.kernel,
        out_type=jax.ShapeDtypeStruct((n_nodes, D), jnp.float32),
        mesh=mesh,
        compiler_params=cp,
        scratch_types=[
            pltpu.VMEM_SHARED((ALLOC, D), jnp.float32),
            pltpu.VMEM((W,), jnp.int32),
            pltpu.VMEM((W,), jnp.int32),
            pltpu.VMEM((W,), jnp.float32),
            pltpu.VMEM((1, G), jnp.int32),
            pltpu.VMEM((1, G), jnp.int32),
            pltpu.VMEM((1, G), jnp.float32),
            pltpu.VMEM((G, D), jnp.float32),
            pltpu.VMEM((64, D), jnp.float32),
        ],
    )
    def seg_kernel(x_hbm, src_hbm, dst_hbm, ew_hbm, out_hbm,
                   acc, wsrc, wdst, wew, csrc, cdst, cew, rows, zbuf):
        c = lax.axis_index("c")
        s = lax.axis_index("s")
        wid = s * NC + c

        zero16f = jnp.zeros((LANES,), jnp.float32)
        iota16 = jnp.arange(LANES, dtype=jnp.int32)
        zero16i = jnp.zeros((LANES,), jnp.int32)

        @pl.loop(0, 64)
        def _(r):
            for l in range(D // LANES):
                zbuf[r, pl.ds(l * LANES, LANES)] = zero16f

        def refill():
            @pl.loop(0, G, step=LANES)
            def _(i):
                csrc[0, pl.ds(i, LANES)] = wid * G + i + iota16
                cdst[0, pl.ds(i, LANES)] = TRASH + (iota16 & 7)
                cew[0, pl.ds(i, LANES)] = zero16f

        def flush():
            pltpu.sync_copy(x_hbm.at[csrc.at[0]], rows)

            @pl.loop(0, G, step=LANES)
            def _(rb):
                rb16 = pl.multiple_of(rb, LANES)
                ew16 = cew[0, pl.ds(rb16, LANES)]
                for k in range(LANES):
                    sval = ew16[k]
                    for l in range(D // LANES):
                        rows[rb16 + k, pl.ds(l * LANES, LANES)] = (
                            rows[rb16 + k, pl.ds(l * LANES, LANES)] * sval)

            pltpu.sync_copy(rows, acc.at[cdst.at[0]], add=True)
            refill()

        for phase in range(chunks_per_core):
            chunk = c * chunks_per_core + phase
            lo = chunk * CHUNK
            hi = jnp.minimum(lo + CHUNK, n_nodes)

            @pl.loop(0, ALLOC // NS // 64)
            def _(k):
                acc_off = pl.multiple_of(s * (ALLOC // NS) + k * 64, 64)
                pltpu.sync_copy(zbuf, acc.at[pl.ds(acc_off, 64)])
            refill()
            plsc.subcore_barrier()

            def win_body(w, count):
                base = pl.multiple_of(s * ept + w * W, W)
                pltpu.sync_copy(src_hbm.at[pl.ds(base, W)], wsrc)
                pltpu.sync_copy(dst_hbm.at[pl.ds(base, W)], wdst)
                pltpu.sync_copy(ew_hbm.at[pl.ds(base, W)], wew)

                def slice_body(i, cnt):
                    off = pl.multiple_of(i * LANES, LANES)
                    sv = wsrc[pl.ds(off, LANES)]
                    dv = wdst[pl.ds(off, LANES)]
                    wv = wew[pl.ds(off, LANES)]
                    m = (dv >= lo) & (dv < hi)
                    mi = m.astype(jnp.int32)
                    cs = plsc.cumsum(mi)
                    pos = cnt + cs - mi
                    plsc.store_scatter(csrc, [zero16i, pos], sv, mask=m)
                    plsc.store_scatter(cdst, [zero16i, pos], dv - lo, mask=m)
                    plsc.store_scatter(cew, [zero16i, pos], wv, mask=m)
                    cnt = cnt + jnp.sum(mi)

                    @pl.when(cnt >= FLUSH_AT)
                    def _():
                        flush()

                    return jnp.where(cnt >= FLUSH_AT, 0, cnt)

                return lax.fori_loop(0, W // LANES, slice_body, count)

            count = lax.fori_loop(0, n_win, win_body, jnp.int32(0))
            flush()
            plsc.subcore_barrier()

            rc = hi - lo
            row0 = pl.multiple_of(s * full_rows, 8)
            pltpu.sync_copy(acc.at[pl.ds(row0, full_rows)],
                            out_hbm.at[pl.ds(lo + row0, full_rows)])
            tail_base = full_rows * NS

            @pl.when(tail_base + s * 8 < rc)
            def _():
                t0 = pl.multiple_of(tail_base + s * 8, 8)
                pltpu.sync_copy(acc.at[pl.ds(t0, 8)],
                                out_hbm.at[pl.ds(lo + t0, 8)])

            plsc.subcore_barrier()

    return seg_kernel(x, src, dst, ew)


def _tc_linear(aggr, x, w_rel_t, w_root_t, b2d):
    n = aggr.shape[0]
    blk = 1000
    grid = (n // blk,)

    def body(a_ref, x_ref, wr_ref, wq_ref, b_ref, o_ref):
        o_ref[...] = (
            jnp.dot(a_ref[...], wr_ref[...], preferred_element_type=jnp.float32)
            + jnp.dot(x_ref[...], wq_ref[...], preferred_element_type=jnp.float32)
            + b_ref[...])

    return pl.pallas_call(
        body,
        grid=grid,
        in_specs=[
            pl.BlockSpec((blk, D), lambda i: (i, 0)),
            pl.BlockSpec((blk, D), lambda i: (i, 0)),
            pl.BlockSpec((D, D), lambda i: (0, 0)),
            pl.BlockSpec((D, D), lambda i: (0, 0)),
            pl.BlockSpec((1, D), lambda i: (0, 0)),
        ],
        out_specs=pl.BlockSpec((blk, D), lambda i: (i, 0)),
        out_shape=jax.ShapeDtypeStruct((n, D), jnp.float32),
    )(aggr, x, w_rel_t, w_root_t, b2d)


def kernel(x, edge_index, edge_weights, W_rel, b_rel, W_root):
    n_nodes = x.shape[0]
    n_elec = 19
    repeat = n_nodes // n_elec
    ew_full = jnp.tile(edge_weights, repeat)

    src = edge_index[0]
    dst = edge_index[1]
    e = src.shape[0]
    e_pad = -(-e // (NS * W)) * (NS * W)
    pad = e_pad - e
    src_p = jnp.concatenate([src, jnp.zeros((pad,), jnp.int32)])
    dst_p = jnp.concatenate([dst, jnp.full((pad,), -1, jnp.int32)])
    ew_p = jnp.concatenate([ew_full, jnp.zeros((pad,), jnp.float32)])

    aggr = _sc_segment_sum(x, src_p, dst_p, ew_p, n_nodes)
    return _tc_linear(aggr, x, W_rel.T, W_root.T, b_rel[None, :])
